# baseline (speedup 1.0000x reference)
import functools

import jax
import jax.numpy as jnp
from jax import lax
from jax.experimental import pallas as pl
from jax.experimental.pallas import tpu as pltpu
from jax.experimental.pallas import tpu_sc as plsc

_NUM_CLASSES = 100000
_HIDDEN = 64
_DROP_P = 0.35
_BATCH = 16384

_LANES = 16
_NUM_WORKERS = 32
_B_PER_W = _BATCH // _NUM_WORKERS
_IDX_CHUNK = 64
_N_CHUNKS = _B_PER_W // _IDX_CHUNK


def _embed_body(labels_hbm, unif_hbm, table_hbm, out_hbm,
                lab_v, u_v, idx_v, rows_v, sem):
    wid = lax.axis_index("s") * 2 + lax.axis_index("c")
    base = wid * _B_PER_W

    pltpu.sync_copy(labels_hbm.at[pl.ds(base, _B_PER_W)], lab_v)
    pltpu.sync_copy(unif_hbm.at[pl.ds(base, _B_PER_W)], u_v)

    cfg_row = jnp.full((_LANES,), _NUM_CLASSES, dtype=jnp.int32)
    thresh = jnp.full((_LANES,), _DROP_P, dtype=jnp.float32)
    for c in range(_N_CHUNKS):
        for i in range(_IDX_CHUNK // _LANES):
            off = c * _IDX_CHUNK + i * _LANES
            lab = lab_v[pl.ds(off, _LANES)]
            u = u_v[pl.ds(off, _LANES)]
            idx_v[c, pl.ds(i * _LANES, _LANES)] = jnp.where(
                u < thresh, cfg_row, lab)

    copies = []
    for c in range(_N_CHUNKS):
        copies.append(pltpu.async_copy(
            table_hbm.at[idx_v.at[c]],
            rows_v.at[pl.ds(c * _IDX_CHUNK, _IDX_CHUNK)],
            sem))
    for cp in copies:
        cp.wait()

    pltpu.sync_copy(rows_v, out_hbm.at[pl.ds(base, _B_PER_W)])


@jax.jit
def _embed(labels, unif, table):
    mesh = plsc.VectorSubcoreMesh(core_axis_name="c", subcore_axis_name="s")
    fn = functools.partial(
        pl.kernel,
        mesh=mesh,
        out_type=jax.ShapeDtypeStruct((_BATCH, _HIDDEN), jnp.float32),
        scratch_types=[
            pltpu.VMEM((_B_PER_W,), jnp.int32),
            pltpu.VMEM((_B_PER_W,), jnp.float32),
            pltpu.VMEM((_N_CHUNKS, _IDX_CHUNK), jnp.int32),
            pltpu.VMEM((_B_PER_W, _HIDDEN), jnp.float32),
            pltpu.SemaphoreType.DMA,
        ],
        compiler_params=pltpu.CompilerParams(use_tc_tiling_on_sc=False),
    )(_embed_body)
    return fn(labels, unif, table)


def kernel(labels, table):
    unif = jax.random.uniform(jax.random.key(42), (labels.shape[0],))
    return _embed(labels, unif, table)


# speedup vs baseline: 1.9705x; 1.9705x over previous
import functools

import jax
import jax.numpy as jnp
from jax import lax
from jax.experimental import pallas as pl
from jax.experimental.pallas import tpu as pltpu
from jax.experimental.pallas import tpu_sc as plsc

_NUM_CLASSES = 100000
_HIDDEN = 64
_DROP_P = 0.35
_BATCH = 16384

_LANES = 16
_NUM_WORKERS = 32
_B_PER_W = _BATCH // _NUM_WORKERS
_IDX_CHUNK = 128
_N_CHUNKS = _B_PER_W // _IDX_CHUNK


def _embed_body(labels_hbm, unif_hbm, table_hbm, out_hbm,
                lab_v, u_v, idx_v, cfg_v, rows_v, sem_g, sem_c):
    wid = lax.axis_index("s") * 2 + lax.axis_index("c")
    base = wid * _B_PER_W

    pltpu.sync_copy(labels_hbm.at[pl.ds(base, _B_PER_W)], lab_v)
    pltpu.sync_copy(unif_hbm.at[pl.ds(base, _B_PER_W)],
                    u_v.at[pl.ds(0, _B_PER_W)])

    for c in range(_N_CHUNKS):
        for i in range(_IDX_CHUNK // _LANES):
            off = c * _IDX_CHUNK + i * _LANES
            idx_v[c, pl.ds(i * _LANES, _LANES)] = lab_v[pl.ds(off, _LANES)]

    copies = [
        pltpu.async_copy(
            table_hbm.at[idx_v.at[c]],
            rows_v.at[pl.ds(c * _IDX_CHUNK, _IDX_CHUNK)],
            sem_g)
        for c in range(_N_CHUNKS)
    ]
    cfg_copy = pltpu.async_copy(
        table_hbm.at[pl.ds(_NUM_CLASSES, 1)], cfg_v, sem_c)

    cfg_copy.wait()
    for cp in copies:
        cp.wait()

    cregs = [cfg_v[0, pl.ds(q * _LANES, _LANES)]
             for q in range(_HIDDEN // _LANES)]

    def _ow(r, carry):
        u_r = u_v[pl.ds(r, _LANES)][0]

        @pl.when(u_r < _DROP_P)
        def _():
            for q in range(_HIDDEN // _LANES):
                rows_v[r, pl.ds(q * _LANES, _LANES)] = cregs[q]

        return carry

    lax.fori_loop(0, _B_PER_W, _ow, jnp.int32(0))

    pltpu.sync_copy(rows_v, out_hbm.at[pl.ds(base, _B_PER_W)])


@jax.jit
def _embed(labels, unif, table):
    mesh = plsc.VectorSubcoreMesh(core_axis_name="c", subcore_axis_name="s")
    fn = functools.partial(
        pl.kernel,
        mesh=mesh,
        out_type=jax.ShapeDtypeStruct((_BATCH, _HIDDEN), jnp.float32),
        scratch_types=[
            pltpu.VMEM((_B_PER_W,), jnp.int32),
            pltpu.VMEM((_B_PER_W + _LANES,), jnp.float32),
            pltpu.VMEM((_N_CHUNKS, _IDX_CHUNK), jnp.int32),
            pltpu.VMEM((1, _HIDDEN), jnp.float32),
            pltpu.VMEM((_B_PER_W, _HIDDEN), jnp.float32),
            pltpu.SemaphoreType.DMA,
            pltpu.SemaphoreType.DMA,
        ],
        compiler_params=pltpu.CompilerParams(use_tc_tiling_on_sc=False),
    )(_embed_body)
    return fn(labels, unif, table)


def kernel(labels, table):
    unif = jax.random.uniform(jax.random.key(42), (labels.shape[0],))
    return _embed(labels, unif, table)
